# Initial kernel scaffold; baseline (speedup 1.0000x reference)
import jax
import jax.numpy as jnp
from jax.experimental import pallas as pl


def kernel(senders, receivers, edge_attr, x, W, b):
    raise NotImplementedError("write your pallas kernel here")


# speedup vs baseline: 1.5650x; 1.5650x over previous
import functools

import jax
import jax.numpy as jnp
from jax import lax
from jax.experimental import pallas as pl
from jax.experimental.pallas import tpu as pltpu
from jax.experimental.pallas import tpu_sc as plsc

E = 320000
N = 10000
R = 16
C = 128


_BE = 2000


def _proj_body(ea_ref, w_ref, b_ref, o_ref):
    o_ref[...] = (
        jnp.dot(ea_ref[...], w_ref[...], preferred_element_type=jnp.float32)
        + b_ref[...]
    )


def _proj_tc(edge_attr, W, b):
    return pl.pallas_call(
        _proj_body,
        grid=(E // _BE,),
        in_specs=[
            pl.BlockSpec((_BE, R), lambda i: (i, 0)),
            pl.BlockSpec((R, C), lambda i: (0, 0)),
            pl.BlockSpec((1, C), lambda i: (0, 0)),
        ],
        out_specs=pl.BlockSpec((_BE, C), lambda i: (i, 0)),
        out_shape=jax.ShapeDtypeStruct((E, C), jnp.float32),
    )(edge_attr, W, b.reshape(1, C))


_K = 200


def _combine_sc(senders, receivers, proj, x):
    info = plsc.get_sparse_core_info()
    nc = info.num_cores
    nw = nc * info.num_subcores
    per_w = E // nw
    n_chunk = per_w // _K

    mesh = plsc.VectorSubcoreMesh(core_axis_name="c", subcore_axis_name="s")

    @functools.partial(
        pl.kernel,
        mesh=mesh,
        out_type=jax.ShapeDtypeStruct((E, C), jnp.float32),
        scratch_types=[
            pltpu.VMEM((_K,), jnp.int32),
            pltpu.VMEM((_K,), jnp.int32),
            pltpu.VMEM((_K, C), jnp.float32),
            pltpu.VMEM((_K, C), jnp.float32),
            pltpu.VMEM((_K, C), jnp.float32),
            pltpu.SemaphoreType.DMA,
            pltpu.SemaphoreType.DMA,
        ],
    )
    def k(s_hbm, r_hbm, proj_hbm, x_hbm, out_hbm,
          sidx, ridx, rows_s, rows_r, projv, sem1, sem2):
        wid = lax.axis_index("s") * nc + lax.axis_index("c")

        def chunk_body(ci, carry):
            base = wid * per_w + ci * _K
            pltpu.sync_copy(s_hbm.at[pl.ds(base, _K)], sidx)
            pltpu.sync_copy(r_hbm.at[pl.ds(base, _K)], ridx)
            cp1 = pltpu.async_copy(x_hbm.at[sidx], rows_s, sem1)
            cp2 = pltpu.async_copy(x_hbm.at[ridx], rows_r, sem2)
            pltpu.sync_copy(proj_hbm.at[pl.ds(base, _K)], projv)
            cp1.wait()
            cp2.wait()

            def edge_body(e, c2):
                for c8 in range(C // 16):
                    sl = pl.ds(c8 * 16, 16)
                    rows_s[e, sl] = (rows_s[e, sl] + rows_r[e, sl]) * projv[e, sl]
                return c2

            lax.fori_loop(0, _K, edge_body, 0, unroll=2)
            pltpu.sync_copy(rows_s, out_hbm.at[pl.ds(base, _K)])
            return carry

        lax.fori_loop(0, n_chunk, chunk_body, 0)

    return k(senders, receivers, proj, x)


def kernel(senders, receivers, edge_attr, x, W, b):
    proj = _proj_tc(edge_attr, W, b)
    return _combine_sc(senders, receivers, proj, x)


# speedup vs baseline: 1.7951x; 1.1471x over previous
import functools

import jax
import jax.numpy as jnp
from jax import lax
from jax.experimental import pallas as pl
from jax.experimental.pallas import tpu as pltpu
from jax.experimental.pallas import tpu_sc as plsc

E = 320000
N = 10000
R = 16
C = 128


_BE = 2000


def _proj_body(ea_ref, w_ref, b_ref, o_ref):
    o_ref[...] = (
        jnp.dot(ea_ref[...], w_ref[...], preferred_element_type=jnp.float32)
        + b_ref[...]
    )


def _proj_tc(edge_attr, W, b):
    return pl.pallas_call(
        _proj_body,
        grid=(E // _BE,),
        in_specs=[
            pl.BlockSpec((_BE, R), lambda i: (i, 0)),
            pl.BlockSpec((R, C), lambda i: (0, 0)),
            pl.BlockSpec((1, C), lambda i: (0, 0)),
        ],
        out_specs=pl.BlockSpec((_BE, C), lambda i: (i, 0)),
        out_shape=jax.ShapeDtypeStruct((E, C), jnp.float32),
    )(edge_attr, W, b.reshape(1, C))


_K = 80
_NBUF = 2


def _combine_sc(senders, receivers, proj, x):
    info = plsc.get_sparse_core_info()
    nc = info.num_cores
    nw = nc * info.num_subcores
    per_w = E // nw
    n_chunk = per_w // _K

    mesh = plsc.VectorSubcoreMesh(core_axis_name="c", subcore_axis_name="s")

    scratch = []
    for _ in range(_NBUF):
        scratch += [
            pltpu.VMEM((_K,), jnp.int32),
            pltpu.VMEM((_K,), jnp.int32),
            pltpu.VMEM((_K, C), jnp.float32),
            pltpu.VMEM((_K, C), jnp.float32),
            pltpu.VMEM((_K, C), jnp.float32),
            pltpu.VMEM((_K, C), jnp.float32),
            pltpu.SemaphoreType.DMA,
            pltpu.SemaphoreType.DMA,
        ]

    @functools.partial(
        pl.kernel,
        mesh=mesh,
        out_type=jax.ShapeDtypeStruct((E, C), jnp.float32),
        scratch_types=scratch,
    )
    def k(s_hbm, r_hbm, proj_hbm, x_hbm, out_hbm, *bufs):
        wid = lax.axis_index("s") * nc + lax.axis_index("c")
        w0 = wid * per_w
        B = [bufs[i * 8:(i + 1) * 8] for i in range(_NBUF)]

        def stage(ci, b):
            sidx, ridx, rows_s, rows_r, projv, _, sem_in, _ = B[b]
            base = w0 + ci * _K
            pltpu.sync_copy(s_hbm.at[pl.ds(base, _K)], sidx)
            pltpu.sync_copy(r_hbm.at[pl.ds(base, _K)], ridx)
            pltpu.async_copy(x_hbm.at[sidx], rows_s, sem_in)
            pltpu.async_copy(x_hbm.at[ridx], rows_r, sem_in)
            pltpu.async_copy(proj_hbm.at[pl.ds(base, _K)], projv, sem_in)

        def wait_in(b):
            sidx, ridx, rows_s, rows_r, projv, _, sem_in, _ = B[b]
            pltpu.make_async_copy(x_hbm.at[sidx], rows_s, sem_in).wait()
            pltpu.make_async_copy(x_hbm.at[ridx], rows_r, sem_in).wait()
            pltpu.make_async_copy(proj_hbm.at[pl.ds(0, _K)], projv, sem_in).wait()

        def drain_out(b):
            outb = B[b][5]
            sem_out = B[b][7]
            pltpu.make_async_copy(outb, out_hbm.at[pl.ds(w0, _K)], sem_out).wait()

        def compute_and_out(ci, b):
            _, _, rows_s, rows_r, projv, outb, _, sem_out = B[b]

            def ebody(e, c2):
                for c8 in range(C // 16):
                    sl = pl.ds(c8 * 16, 16)
                    outb[e, sl] = (rows_s[e, sl] + rows_r[e, sl]) * projv[e, sl]
                return c2

            lax.fori_loop(0, _K, ebody, 0, unroll=2)
            base = w0 + ci * _K
            pltpu.async_copy(outb, out_hbm.at[pl.ds(base, _K)], sem_out)

        stage(0, 0)
        stage(1, 1)

        def outer(g, carry):
            for b in range(_NBUF):
                ci = g * _NBUF + b
                wait_in(b)

                @pl.when(ci >= _NBUF)
                def _():
                    drain_out(b)

                compute_and_out(ci, b)

                @pl.when(ci + _NBUF < n_chunk)
                def _():
                    stage(ci + _NBUF, b)
            return carry

        lax.fori_loop(0, (n_chunk - 1) // _NBUF, outer, 0)

        ci_last = n_chunk - 1
        bl = ci_last % _NBUF
        wait_in(bl)
        drain_out(bl)
        compute_and_out(ci_last, bl)
        drain_out(bl)
        drain_out(1 - bl)

    return k(senders, receivers, proj, x)


def kernel(senders, receivers, edge_attr, x, W, b):
    proj = _proj_tc(edge_attr, W, b)
    return _combine_sc(senders, receivers, proj, x)
